# baseline (speedup 1.0000x reference)
import functools

import jax
import jax.numpy as jnp
from jax import lax
from jax.experimental import pallas as pl
from jax.experimental.pallas import tpu as pltpu
from jax.experimental.pallas import tpu_sc as plsc

N = 10000
E = 160000
D = 256
H = 256

NC = 2
NS = 16
L = 16

NT = N
CH = 80
EPT = E // NS
NCHUNK = EPT // CH
EPW = E // (NC * NS)
NCHUNK_D = 63
EPW_BUF = 5008
RZ = 624
RZL = 648
RCL = 640
ACC_R = 15 * RZ + RZL

_mesh = plsc.VectorSubcoreMesh(
    core_axis_name="c", subcore_axis_name="s", num_cores=NC, num_subcores=NS
)


ZR = 24


def _zero_acc(zbuf, zsem, acc, t):
    for r in range(ZR):
        for l in range(H // NC // L):
            zbuf[r, pl.ds(l * L, L)] = jnp.zeros((L,), jnp.float32)

    nz = jnp.where(t == 15, RZL // ZR, RZ // ZR)

    def body(j, carry):
        @pl.when(j >= 8)
        def _():
            pltpu.make_async_copy(zbuf, acc.at[pl.ds(0, ZR)], zsem).wait()

        pltpu.async_copy(zbuf, acc.at[pl.ds(t * RZ + j * ZR, ZR)], zsem)
        return carry

    lax.fori_loop(0, nz, body, 0)
    for _ in range(8):
        pltpu.make_async_copy(zbuf, acc.at[pl.ds(0, ZR)], zsem).wait()


def _copy_out(acc, out0_hbm, out1_hbm, c, t):
    for ci, o in ((0, out0_hbm), (1, out1_hbm)):
        @pl.when(jnp.logical_and(c == ci, t < 15))
        def _(o=o):
            pltpu.sync_copy(acc.at[pl.ds(t * RZ, RZ)], o.at[pl.ds(t * RZ, RZ)])

        @pl.when(jnp.logical_and(c == ci, t == 15))
        def _(o=o):
            pltpu.sync_copy(acc.at[pl.ds(15 * RZ, RCL)],
                            o.at[pl.ds(15 * RZ, RCL)])


def _deg_body(src_hbm, dst_hbm, ones_hbm, out0_hbm, out1_hbm,
              acc, src_all, dst_all, srcm2, ones_v, zbuf, sem, zsem):
    c = lax.axis_index("c")
    t = lax.axis_index("s")
    base = (c * NS + t) * EPW
    pltpu.sync_copy(src_hbm.at[pl.ds(base, EPW)], src_all.at[pl.ds(0, EPW)])
    pltpu.sync_copy(dst_hbm.at[pl.ds(base, EPW)], dst_all.at[pl.ds(0, EPW)])

    def mask_body(j, carry):
        for l in range(CH // L):
            s = src_all[pl.ds(j * CH + l * L, L)]
            d = dst_all[pl.ds(j * CH + l * L, L)]
            srcm2[j, pl.ds(l * L, L)] = jnp.where(s == d, NT, s)
        return carry

    lax.fori_loop(0, NCHUNK_D - 1, mask_body, 0)
    jt = NCHUNK_D - 1
    for l in range(CH // L):
        if l < 2:
            s = src_all[pl.ds(jt * CH + l * L, L)]
            d = dst_all[pl.ds(jt * CH + l * L, L)]
            srcm2[jt, pl.ds(l * L, L)] = jnp.where(s == d, NT, s)
        elif l == 2:
            s = src_all[pl.ds(jt * CH + l * L, L)]
            d = dst_all[pl.ds(jt * CH + l * L, L)]
            valid = lax.iota(jnp.int32, L) < (EPW - jt * CH - l * L)
            keep = jnp.logical_and(valid, s != d)
            srcm2[jt, pl.ds(l * L, L)] = jnp.where(keep, s, NT)
        else:
            srcm2[jt, pl.ds(l * L, L)] = jnp.full((L,), NT, jnp.int32)
    pltpu.sync_copy(ones_hbm, ones_v)
    _zero_acc(zbuf, zsem, acc, t)
    plsc.subcore_barrier()

    def body(j, carry):
        @pl.when(j >= 4)
        def _():
            pltpu.make_async_copy(ones_v, acc.at[srcm2.at[0]], sem).wait()

        pltpu.async_copy(ones_v, acc.at[srcm2.at[j]], sem, add=True)
        return carry

    lax.fori_loop(0, NCHUNK_D, body, 0)
    for _ in range(4):
        pltpu.make_async_copy(ones_v, acc.at[srcm2.at[0]], sem).wait()
    plsc.subcore_barrier()
    _copy_out(acc, out0_hbm, out1_hbm, c, t)


_deg_call = pl.kernel(
    _deg_body,
    out_type=(jax.ShapeDtypeStruct((N, H // NC), jnp.float32),
              jax.ShapeDtypeStruct((N, H // NC), jnp.float32)),
    mesh=_mesh,
    scratch_types=[
        pltpu.VMEM_SHARED((ACC_R, H // NC), jnp.float32),
        pltpu.VMEM((EPW_BUF,), jnp.int32),
        pltpu.VMEM((EPW_BUF,), jnp.int32),
        pltpu.VMEM((NCHUNK_D, CH), jnp.int32),
        pltpu.VMEM((CH, H // NC), jnp.float32),
        pltpu.VMEM((ZR, H // NC), jnp.float32),
        pltpu.SemaphoreType.DMA,
        pltpu.SemaphoreType.DMA,
    ],
)


CHF = 80
NCHF = EPT // CHF
NBUF = 4
NCH_MAIN = NCHF - 1


def _feat_body(ut_hbm, src_hbm, dst_hbm, out0_hbm, out1_hbm,
               acc, src2, dst2, dstm2, rbs, zbuf, isem, gsem, ssem, zsem):
    c = lax.axis_index("c")
    t = lax.axis_index("s")
    base = t * EPT
    uc = ut_hbm.at[c]

    def start_i(j, slot):
        pltpu.async_copy(src_hbm.at[pl.ds(base + j * CHF, CHF)],
                         src2.at[slot], isem)
        pltpu.async_copy(dst_hbm.at[pl.ds(base + j * CHF, CHF)],
                         dst2.at[slot], isem)

    def wait_i():
        pltpu.make_async_copy(src_hbm.at[pl.ds(0, CHF)], src2.at[0],
                              isem).wait()
        pltpu.make_async_copy(dst_hbm.at[pl.ds(0, CHF)], dst2.at[0],
                              isem).wait()

    def start_g(slot):
        pltpu.async_copy(uc.at[src2.at[slot]], rbs[slot], gsem)

    def wait_g(slot):
        pltpu.make_async_copy(uc.at[src2.at[slot]], rbs[slot], gsem).wait()

    def start_s(slot):
        pltpu.async_copy(rbs[slot], acc.at[dstm2.at[slot]], ssem, add=True)

    def wait_s(slot):
        pltpu.make_async_copy(rbs[slot], acc.at[dstm2.at[slot]], ssem).wait()

    def compute_mask(slot):
        for l in range(CHF // L):
            s = src2[slot, pl.ds(l * L, L)]
            d = dst2[slot, pl.ds(l * L, L)]
            dstm2[slot, pl.ds(l * L, L)] = jnp.where(s == d, NT, d)

    start_i(0, 0)
    start_i(1, 1)
    _zero_acc(zbuf, zsem, acc, t)
    wait_i()
    wait_i()
    start_g(0)
    start_g(1)
    start_i(2, 2)
    plsc.subcore_barrier()

    def outer(ko, carry):
        for b in range(NBUF):
            j = ko * NBUF + b

            @pl.when(j >= 2)
            def _():
                wait_s((b - 2) % NBUF)

            @pl.when(j + 2 < NCHF)
            def _():
                wait_i()
                start_g((b + 2) % NBUF)

            @pl.when(j + 3 < NCHF)
            def _():
                start_i(j + 3, (b + 3) % NBUF)

            compute_mask(b)
            wait_g(b)
            start_s(b)
        return carry

    lax.fori_loop(0, NCH_MAIN // NBUF, outer, 0)
    compute_mask(0)
    wait_g(0)
    wait_s(2)
    wait_s(3)
    start_s(0)
    wait_s(0)
    plsc.subcore_barrier()
    _copy_out(acc, out0_hbm, out1_hbm, c, t)


_feat_call = pl.kernel(
    _feat_body,
    out_type=(jax.ShapeDtypeStruct((N, H // NC), jnp.float32),
              jax.ShapeDtypeStruct((N, H // NC), jnp.float32)),
    mesh=_mesh,
    scratch_types=[
        pltpu.VMEM_SHARED((ACC_R, H // NC), jnp.float32),
        pltpu.VMEM((NBUF, CHF), jnp.int32),
        pltpu.VMEM((NBUF, CHF), jnp.int32),
        pltpu.VMEM((NBUF, CHF), jnp.int32),
        [pltpu.VMEM((CHF, H // NC), jnp.float32) for _ in range(NBUF)],
        pltpu.VMEM((ZR, H // NC), jnp.float32),
        pltpu.SemaphoreType.DMA,
        pltpu.SemaphoreType.DMA,
        pltpu.SemaphoreType.DMA,
        pltpu.SemaphoreType.DMA,
    ],
)


BN = 1000
_PREC = None


def _tc_a0_body(x_ref, w10_ref, w11_ref, b1_ref, p1_ref, xw1_ref):
    xv = x_ref[...]
    p1_ref[...] = (
        jnp.dot(xv, w10_ref[...], preferred_element_type=jnp.float32,
                precision=_PREC) + b1_ref[...]
    )
    xw1_ref[...] = jnp.dot(xv, w11_ref[...], preferred_element_type=jnp.float32,
                           precision=_PREC)


def _tc_a0(x, w10, w11, b1):
    return pl.pallas_call(
        _tc_a0_body,
        grid=(N // BN,),
        in_specs=[
            pl.BlockSpec((BN, D), lambda i: (i, 0)),
            pl.BlockSpec((D, H), lambda i: (0, 0)),
            pl.BlockSpec((D, H), lambda i: (0, 0)),
            pl.BlockSpec((1, H), lambda i: (0, 0)),
        ],
        out_specs=[
            pl.BlockSpec((BN, H), lambda i: (i, 0)),
            pl.BlockSpec((BN, H), lambda i: (i, 0)),
        ],
        out_shape=[
            jax.ShapeDtypeStruct((N, H), jnp.float32),
            jax.ShapeDtypeStruct((N, H), jnp.float32),
        ],
    )(x, w10, w11, b1)


def _tc_a1_body(xw1_ref, d0_ref, d1_ref, u1t_ref, dis_ref):
    deg = d0_ref[...][:, 0:L] + d1_ref[...][:, 0:L]
    dis = jnp.where(deg > 0, jax.lax.rsqrt(deg), 0.0)
    dis_ref[...] = dis
    u = xw1_ref[...] * dis[:, 0:1]
    u1t_ref[0] = u[:, : H // NC]
    u1t_ref[1] = u[:, H // NC:]


def _tc_a1(xw1, d0, d1):
    return pl.pallas_call(
        _tc_a1_body,
        grid=(N // BN,),
        in_specs=[
            pl.BlockSpec((BN, H), lambda i: (i, 0)),
            pl.BlockSpec((BN, H // NC), lambda i: (i, 0)),
            pl.BlockSpec((BN, H // NC), lambda i: (i, 0)),
        ],
        out_specs=[
            pl.BlockSpec((NC, BN, H // NC), lambda i: (0, i, 0)),
            pl.BlockSpec((BN, L), lambda i: (i, 0)),
        ],
        out_shape=[
            jax.ShapeDtypeStruct((NC, N, H // NC), jnp.float32),
            jax.ShapeDtypeStruct((N, L), jnp.float32),
        ],
    )(xw1, d0, d1)


def _tc_ba_body(p1_ref, s10_ref, s11_ref, dis_ref, g_ref, bb_ref,
                w21_ref, h_ref, u2t_ref):
    d = dis_ref[...][:, 0:1]
    s1 = jnp.concatenate([s10_ref[...], s11_ref[...]], axis=1)
    h = p1_ref[...] - d * s1
    mu = jnp.mean(h, axis=1, keepdims=True)
    var = jnp.mean((h - mu) ** 2, axis=1, keepdims=True)
    h = (h - mu) * jax.lax.rsqrt(var + 1e-5) * g_ref[...] + bb_ref[...]
    h = jnp.where(h > 0, h, 0.01 * h)
    h_ref[...] = h
    u = jnp.dot(h, w21_ref[...], preferred_element_type=jnp.float32,
                precision=_PREC) * d
    u2t_ref[0] = u[:, : H // NC]
    u2t_ref[1] = u[:, H // NC:]


def _tc_ba(p1, s10, s11, dis, g, bb, w21):
    return pl.pallas_call(
        _tc_ba_body,
        grid=(N // BN,),
        in_specs=[
            pl.BlockSpec((BN, H), lambda i: (i, 0)),
            pl.BlockSpec((BN, H // NC), lambda i: (i, 0)),
            pl.BlockSpec((BN, H // NC), lambda i: (i, 0)),
            pl.BlockSpec((BN, L), lambda i: (i, 0)),
            pl.BlockSpec((1, H), lambda i: (0, 0)),
            pl.BlockSpec((1, H), lambda i: (0, 0)),
            pl.BlockSpec((H, H), lambda i: (0, 0)),
        ],
        out_specs=[
            pl.BlockSpec((BN, H), lambda i: (i, 0)),
            pl.BlockSpec((NC, BN, H // NC), lambda i: (0, i, 0)),
        ],
        out_shape=[
            jax.ShapeDtypeStruct((N, H), jnp.float32),
            jax.ShapeDtypeStruct((NC, N, H // NC), jnp.float32),
        ],
    )(p1, s10, s11, dis, g, bb, w21)


def _tc_bb_body(h_ref, w20_ref, b2_ref, p2_ref):
    p2_ref[...] = (
        jnp.dot(h_ref[...], w20_ref[...], preferred_element_type=jnp.float32,
                precision=_PREC) + b2_ref[...]
    )


def _tc_bb(h, w20, b2):
    return pl.pallas_call(
        _tc_bb_body,
        grid=(N // BN,),
        in_specs=[
            pl.BlockSpec((BN, H), lambda i: (i, 0)),
            pl.BlockSpec((H, H), lambda i: (0, 0)),
            pl.BlockSpec((1, H), lambda i: (0, 0)),
        ],
        out_specs=pl.BlockSpec((BN, H), lambda i: (i, 0)),
        out_shape=jax.ShapeDtypeStruct((N, H), jnp.float32),
    )(h, w20, b2)


def _tc_c_body(p2_ref, s20_ref, s21_ref, dis_ref, out_ref):
    d = dis_ref[...][:, 0:1]
    s2 = jnp.concatenate([s20_ref[...], s21_ref[...]], axis=1)
    out_ref[...] = p2_ref[...] - d * s2


def _tc_c(p2, s20, s21, dis):
    return pl.pallas_call(
        _tc_c_body,
        grid=(N // BN,),
        in_specs=[
            pl.BlockSpec((BN, H), lambda i: (i, 0)),
            pl.BlockSpec((BN, H // NC), lambda i: (i, 0)),
            pl.BlockSpec((BN, H // NC), lambda i: (i, 0)),
            pl.BlockSpec((BN, L), lambda i: (i, 0)),
        ],
        out_specs=pl.BlockSpec((BN, H), lambda i: (i, 0)),
        out_shape=jax.ShapeDtypeStruct((N, H), jnp.float32),
    )(p2, s20, s21, dis)


@jax.jit
def kernel(x, edge_index, W1_0, W1_1, b1, W2_0, W2_1, b2, ln_g, ln_b):
    src = edge_index[0]
    dst = edge_index[1]
    ones128 = jnp.ones((CH, H // NC), jnp.float32)

    deg0, deg1 = _deg_call(src, dst, ones128)
    p1, xw1 = _tc_a0(x, W1_0, W1_1, b1.reshape(1, H))
    u1t, dis = _tc_a1(xw1, deg0, deg1)
    s10, s11 = _feat_call(u1t, src, dst)
    h, u2t = _tc_ba(p1, s10, s11, dis, ln_g.reshape(1, H),
                    ln_b.reshape(1, H), W2_1)
    s20, s21 = _feat_call(u2t, src, dst)
    p2 = _tc_bb(h, W2_0, b2.reshape(1, H))
    return _tc_c(p2, s20, s21, dis)
